# baseline (speedup 1.0000x reference)
import jax
import jax.numpy as jnp
from jax.experimental import pallas as pl
from jax.experimental.pallas import tpu as pltpu

_BN = 12800
_NFULL = 7
_TAIL = 10400
_NIN = 2
_NOUT = 3


def _dot(wt_ref, x):
    return jax.lax.dot_general(
        wt_ref[...],
        x,
        dimension_numbers=(((0,), (0,)), ((), ())),
        preferred_element_type=jnp.float32,
    )


def _body(wt_ref, ft_any, o_any, ft_v, o_v, ft_t, o_t,
          in_sem, out_sem, tin_sem, tout_sem):
    def in_copy(i, b):
        return pltpu.make_async_copy(
            ft_any.at[:, pl.ds(i * _BN, _BN)], ft_v.at[b], in_sem.at[b])

    def out_copy(i, b):
        return pltpu.make_async_copy(
            o_v.at[b], o_any.at[:, pl.ds(i * _BN, _BN)], out_sem.at[b])

    tail_in = pltpu.make_async_copy(
        ft_any.at[:, pl.ds(_NFULL * _BN, _TAIL)], ft_t, tin_sem)
    tail_out = pltpu.make_async_copy(
        o_t, o_any.at[:, pl.ds(_NFULL * _BN, _TAIL)], tout_sem)

    in_copy(0, 0).start()
    in_copy(1, 1).start()
    tail_in.start()
    for i in range(_NFULL):
        bi = i % _NIN
        bo = i % _NOUT
        in_copy(i, bi).wait()
        if i >= _NOUT:
            out_copy(i - _NOUT, bo).wait()
        o_v[bo] = _dot(wt_ref, ft_v[bi])
        if i + _NIN < _NFULL:
            in_copy(i + _NIN, bi).start()
        out_copy(i, bo).start()
    tail_in.wait()
    o_t[...] = _dot(wt_ref, ft_t[...])
    tail_out.start()
    for i in range(_NFULL - _NOUT, _NFULL):
        out_copy(i, i % _NOUT).wait()
    tail_out.wait()


def kernel(features, W_fc):
    n, k = features.shape
    h = W_fc.shape[0]
    ft = features.T
    wt = W_fc.T
    out_t = pl.pallas_call(
        _body,
        in_specs=[
            pl.BlockSpec((k, h), lambda: (0, 0)),
            pl.BlockSpec(memory_space=pl.ANY),
        ],
        out_specs=pl.BlockSpec(memory_space=pl.ANY),
        out_shape=jax.ShapeDtypeStruct((h, n), jnp.float32),
        scratch_shapes=[
            pltpu.VMEM((_NIN, k, _BN), jnp.float32),
            pltpu.VMEM((_NOUT, h, _BN), jnp.float32),
            pltpu.VMEM((k, _TAIL), jnp.float32),
            pltpu.VMEM((h, _TAIL), jnp.float32),
            pltpu.SemaphoreType.DMA((_NIN,)),
            pltpu.SemaphoreType.DMA((_NOUT,)),
            pltpu.SemaphoreType.DMA,
            pltpu.SemaphoreType.DMA,
        ],
    )(wt, ft)
    return out_t.T
